# speedup vs baseline: 4.5053x; 4.5053x over previous
import functools

import jax
import jax.numpy as jnp
from jax import lax
from jax.experimental import pallas as pl
from jax.experimental.pallas import tpu as pltpu, tpu_sc as plsc


def _make_gather(V, D, B):
  info = plsc.get_sparse_core_info()
  NC, NS = info.num_cores, info.num_subcores
  NW = NC * NS
  assert B % NW == 0
  b_per_w = B // NW
  C = 200
  assert b_per_w % C == 0
  n_chunks = b_per_w // C

  mesh = plsc.VectorSubcoreMesh(core_axis_name="c", subcore_axis_name="s")

  @functools.partial(
      pl.kernel,
      mesh=mesh,
      out_type=jax.ShapeDtypeStruct((B, D), jnp.float32),
      scratch_types=[
          pltpu.VMEM((C,), jnp.int32),
          pltpu.VMEM((C, D), jnp.float32),
          pltpu.SemaphoreType.DMA,
      ],
  )
  def k(table_hbm, idx_hbm, out_hbm, idx_v, rows_v, sem):
    wid = lax.axis_index("s") * NC + lax.axis_index("c")
    base = wid * b_per_w

    def body(c, carry):
      off = base + c * C
      pltpu.sync_copy(idx_hbm.at[pl.ds(off, C)], idx_v)
      pltpu.async_copy(table_hbm.at[idx_v], rows_v, sem).wait()
      pltpu.sync_copy(rows_v, out_hbm.at[pl.ds(off, C)])
      return carry

    lax.fori_loop(0, n_chunks, body, 0)

  return k


def kernel(x, index):
  V, D = x.shape
  B = index.shape[0]
  return _make_gather(V, D, B)(x, index.astype(jnp.int32))


# speedup vs baseline: 7.0009x; 1.5539x over previous
import functools

import jax
import jax.numpy as jnp
from jax import lax
from jax.experimental import pallas as pl
from jax.experimental.pallas import tpu as pltpu, tpu_sc as plsc


def _make_gather(V, D, B):
  info = plsc.get_sparse_core_info()
  NC, NS = info.num_cores, info.num_subcores
  NW = NC * NS
  assert B % NW == 0
  b_per_w = B // NW
  C = 200
  NBUF = 5
  K = 3
  assert b_per_w % C == 0
  n_chunks = b_per_w // C
  assert n_chunks % NBUF == 0
  n_rounds = n_chunks // NBUF

  mesh = plsc.VectorSubcoreMesh(core_axis_name="c", subcore_axis_name="s")

  scratch = ([pltpu.VMEM((C,), jnp.int32)] * NBUF
             + [pltpu.VMEM((C, D), jnp.float32)] * NBUF
             + [pltpu.SemaphoreType.DMA] * (2 * NBUF))

  @functools.partial(
      pl.kernel,
      mesh=mesh,
      out_type=jax.ShapeDtypeStruct((B, D), jnp.float32),
      scratch_types=scratch,
  )
  def k(table_hbm, idx_hbm, out_hbm, *scr):
    idx_v = scr[:NBUF]
    rows_v = scr[NBUF:2 * NBUF]
    gsem = scr[2 * NBUF:3 * NBUF]
    wsem = scr[3 * NBUF:4 * NBUF]
    wid = lax.axis_index("s") * NC + lax.axis_index("c")
    base = wid * b_per_w

    def fire_gather(b, j):
      off = base + j * C
      pltpu.sync_copy(idx_hbm.at[pl.ds(off, C)], idx_v[b])
      pltpu.async_copy(table_hbm.at[idx_v[b]], rows_v[b], gsem[b])

    def wait_gather(b):
      pltpu.make_async_copy(table_hbm.at[idx_v[b]], rows_v[b], gsem[b]).wait()

    def wait_write(b):
      pltpu.make_async_copy(rows_v[b], out_hbm.at[pl.ds(0, C)], wsem[b]).wait()

    for j in range(K):
      fire_gather(j, j)

    def round_body(i, carry):
      for b in range(NBUF):
        j = i * NBUF + b
        bp = (b + K) % NBUF
        @pl.when(jnp.logical_and(j >= NBUF - K, j + K < n_chunks))
        def _():
          wait_write(bp)

        @pl.when(j + K < n_chunks)
        def _():
          fire_gather(bp, j + K)

        wait_gather(b)
        off = base + j * C
        pltpu.async_copy(rows_v[b], out_hbm.at[pl.ds(off, C)], wsem[b])
      return carry

    lax.fori_loop(0, n_rounds, round_body, 0)

    for b in range(NBUF):
      wait_write(b)

  return k


def kernel(x, index):
  V, D = x.shape
  B = index.shape[0]
  return _make_gather(V, D, B)(x, index.astype(jnp.int32))
